# baseline (speedup 1.0000x reference)
import jax
import jax.numpy as jnp
from jax.experimental import pallas as pl


def _body(row_ref, col_ref, o_ref, *, h, w, d):
    ct = col_ref[0:w, :]
    rt = row_ref[0:h, :]
    ctT = ct.T
    rtT = rt.T
    o_ref[0, 0:d] = jnp.broadcast_to(ctT[:, None, :], (d, h, w))
    o_ref[0, d:2 * d] = jnp.broadcast_to(rtT[:, :, None], (d, h, w))


def kernel(x, row_embed, col_embed):
    b = x.shape[0]
    h, w = x.shape[-2], x.shape[-1]
    d = row_embed.shape[1]
    import functools
    body = functools.partial(_body, h=h, w=w, d=d)
    return pl.pallas_call(
        body,
        grid=(b,),
        in_specs=[
            pl.BlockSpec(row_embed.shape, lambda i: (0, 0)),
            pl.BlockSpec(col_embed.shape, lambda i: (0, 0)),
        ],
        out_specs=pl.BlockSpec((1, 2 * d, h, w), lambda i: (i, 0, 0, 0)),
        out_shape=jax.ShapeDtypeStruct((b, 2 * d, h, w), jnp.float32),
    )(row_embed, col_embed)


# speedup vs baseline: 1.0014x; 1.0014x over previous
import functools

import jax
import jax.numpy as jnp
from jax.experimental import pallas as pl
from jax.experimental.pallas import tpu as pltpu


def _body(row_ref, col_ref, o_hbm, plane, sems, *, b, h, w, d):
    ct = col_ref[0:w, :]
    rt = row_ref[0:h, :]
    ctT = ct.T
    rtT = rt.T
    plane[0:d] = jnp.broadcast_to(ctT[:, None, :], (d, h, w))
    plane[d:2 * d] = jnp.broadcast_to(rtT[:, :, None], (d, h, w))
    for i in range(b):
        pltpu.make_async_copy(plane, o_hbm.at[i], sems.at[i]).start()
    for i in range(b):
        pltpu.make_async_copy(plane, o_hbm.at[i], sems.at[i]).wait()


def kernel(x, row_embed, col_embed):
    b = x.shape[0]
    h, w = x.shape[-2], x.shape[-1]
    d = row_embed.shape[1]
    body = functools.partial(_body, b=b, h=h, w=w, d=d)
    return pl.pallas_call(
        body,
        in_specs=[
            pl.BlockSpec(memory_space=pltpu.MemorySpace.VMEM),
            pl.BlockSpec(memory_space=pltpu.MemorySpace.VMEM),
        ],
        out_specs=pl.BlockSpec(memory_space=pltpu.MemorySpace.HBM),
        out_shape=jax.ShapeDtypeStruct((b, 2 * d, h, w), jnp.float32),
        scratch_shapes=[
            pltpu.VMEM((2 * d, h, w), jnp.float32),
            pltpu.SemaphoreType.DMA((b,)),
        ],
    )(row_embed, col_embed)


# speedup vs baseline: 2.7079x; 2.7041x over previous
import functools

import jax
import jax.numpy as jnp
from jax import lax
from jax.experimental import pallas as pl
from jax.experimental.pallas import tpu as pltpu


def _body(row_ref, col_ref, o_hbm, plane, sems, *, b, h, w, d):
    col = col_ref[0:w, :]
    row = row_ref[0:h, :]
    lane = lax.broadcasted_iota(jnp.int32, (w, h * w), 1)
    sub = lax.broadcasted_iota(jnp.int32, (w, h * w), 0)
    t_sel = (lane % w == sub).astype(jnp.float32)
    e_sel = (lane // w == sub).astype(jnp.float32)
    dn = (((0,), (0,)), ((), ()))
    plane[0:d] = lax.dot_general(col, t_sel, dn,
                                 preferred_element_type=jnp.float32)
    plane[d:2 * d] = lax.dot_general(row, e_sel, dn,
                                     preferred_element_type=jnp.float32)
    for i in range(b):
        pltpu.make_async_copy(plane, o_hbm.at[i], sems.at[i]).start()
    for i in range(b):
        pltpu.make_async_copy(plane, o_hbm.at[i], sems.at[i]).wait()


def kernel(x, row_embed, col_embed):
    b = x.shape[0]
    h, w = x.shape[-2], x.shape[-1]
    d = row_embed.shape[1]
    body = functools.partial(_body, b=b, h=h, w=w, d=d)
    out = pl.pallas_call(
        body,
        in_specs=[
            pl.BlockSpec(memory_space=pltpu.MemorySpace.VMEM),
            pl.BlockSpec(memory_space=pltpu.MemorySpace.VMEM),
        ],
        out_specs=pl.BlockSpec(memory_space=pltpu.MemorySpace.HBM),
        out_shape=jax.ShapeDtypeStruct((b, 2 * d, h * w), jnp.float32),
        scratch_shapes=[
            pltpu.VMEM((2 * d, h * w), jnp.float32),
            pltpu.SemaphoreType.DMA((b,)),
        ],
    )(row_embed, col_embed)
    return out.reshape(b, 2 * d, h, w)


# speedup vs baseline: 9.6289x; 3.5559x over previous
import functools

import jax
import jax.numpy as jnp
from jax.experimental import pallas as pl
from jax.experimental.pallas import tpu as pltpu


def _body(row_ref, col_ref, o_hbm, plane, sems, *, b, h, w, d):
    col = col_ref[0:w, :]
    row = row_ref[0:h, :]
    plane[:, :, 0:d] = jnp.broadcast_to(col[None, :, :], (h, w, d))
    plane[:, :, d:2 * d] = jnp.broadcast_to(row[:, None, :], (h, w, d))
    for i in range(b):
        pltpu.make_async_copy(plane, o_hbm.at[i], sems.at[i]).start()
    for i in range(b):
        pltpu.make_async_copy(plane, o_hbm.at[i], sems.at[i]).wait()


def kernel(x, row_embed, col_embed):
    b = x.shape[0]
    h, w = x.shape[-2], x.shape[-1]
    d = row_embed.shape[1]
    body = functools.partial(_body, b=b, h=h, w=w, d=d)
    out = pl.pallas_call(
        body,
        in_specs=[
            pl.BlockSpec(memory_space=pltpu.MemorySpace.VMEM),
            pl.BlockSpec(memory_space=pltpu.MemorySpace.VMEM),
        ],
        out_specs=pl.BlockSpec(memory_space=pltpu.MemorySpace.HBM),
        out_shape=jax.ShapeDtypeStruct((b, h, w, 2 * d), jnp.float32),
        scratch_shapes=[
            pltpu.VMEM((h, w, 2 * d), jnp.float32),
            pltpu.SemaphoreType.DMA((b,)),
        ],
    )(row_embed, col_embed)
    return jnp.transpose(out, (0, 3, 1, 2))
